# baseline (speedup 1.0000x reference)
import jax
import jax.numpy as jnp
from jax.experimental import pallas as pl
from jax.experimental.pallas import tpu as pltpu

IN_F = 4096
OUT_F = 1024
STRIDE = 4
BR = 256


def _tc_body(x_ref, o_ref):
    n = x_ref.shape[0]
    lane = jax.lax.broadcasted_iota(jnp.int32, (n, 128), 1)
    idx = (lane % 32) * STRIDE
    y = jnp.take_along_axis(x_ref[...], idx, axis=1)
    o_ref[...] = y[:, :32]


def kernel(input):
    B, S, F = input.shape
    R = B * S
    x = input.reshape(R * 32, 128)
    out = pl.pallas_call(
        _tc_body,
        grid=(R // BR,),
        in_specs=[pl.BlockSpec((BR * 32, 128), lambda i: (i, 0))],
        out_specs=pl.BlockSpec((BR * 32, 32), lambda i: (i, 0)),
        out_shape=jax.ShapeDtypeStruct((R * 32, 32), jnp.float32),
    )(x)
    return out.reshape(B, S, OUT_F)
